# baseline (speedup 1.0000x reference)
import functools

import jax
import jax.numpy as jnp
from jax import lax
from jax.experimental import pallas as pl
from jax.experimental.pallas import tpu as pltpu
from jax.experimental.pallas import tpu_sc as plsc

V = 1_000_000
B = 16384
L = 50
D = 64
LANES = 16
NC = 2
NS = 16
NW = NC * NS

COLS = 128
FLAT = COLS * D
VFULL = (V // COLS) * COLS
NGRP = VFULL // COLS
NITER = 246
VTAIL = V - VFULL

BPW = B // NW
GB = 8
GPI = GB * L
NG = BPW // GB

_mesh = plsc.VectorSubcoreMesh(core_axis_name="c", subcore_axis_name="s")


@functools.partial(
    pl.kernel,
    mesh=_mesh,
    out_type=jax.ShapeDtypeStruct((V * D,), jnp.float32),
    scratch_types=[
        pltpu.VMEM((D, COLS), jnp.float32),
        pltpu.VMEM((D, COLS), jnp.float32),
        pltpu.VMEM((FLAT,), jnp.float32),
        pltpu.VMEM((FLAT,), jnp.float32),
        pltpu.SemaphoreType.DMA,
        pltpu.SemaphoreType.DMA,
        pltpu.SemaphoreType.DMA,
        pltpu.SemaphoreType.DMA,
    ],
    compiler_params=pltpu.CompilerParams(
        use_tc_tiling_on_sc=True, needs_layout_passes=False),
)
def _relayout(tt_hbm, tail_hbm, out_hbm, in0, in1, fb0, fb1, si0, si1, so0, so1):
    wid = lax.axis_index("s") * NC + lax.axis_index("c")

    def grp(k):
        return jnp.minimum(wid + k * NW, NGRP - 1)

    def in_copy(g, buf, sem):
        return pltpu.make_async_copy(
            tt_hbm.at[:, pl.ds(g * COLS, COLS)], buf, sem)

    def out_copy(g, fb, sem):
        return pltpu.make_async_copy(
            fb, out_hbm.at[pl.ds(g * FLAT, FLAT)], sem)

    pats = [(lax.iota(jnp.int32, LANES) + ch * LANES) * D
            for ch in range(COLS // LANES)]

    def transpose(buf, fb):
        def rbody(r, carry):
            for ch in range(COLS // LANES):
                val = buf[r, pl.ds(ch * LANES, LANES)]
                plsc.store_scatter(fb, [pats[ch] + r], val)
            return carry
        lax.fori_loop(0, D, rbody, 0)

    in_copy(grp(0), in0, si0).start()
    in_copy(grp(1), in1, si1).start()

    def body(i, carry):
        k = 2 * i
        g0, g1, g2, g3 = grp(k), grp(k + 1), grp(k + 2), grp(k + 3)
        in_copy(g0, in0, si0).wait()

        @pl.when(k >= 2)
        def _():
            out_copy(grp(k - 2), fb0, so0).wait()
        transpose(in0, fb0)
        out_copy(g0, fb0, so0).start()

        @pl.when(k + 2 < NITER)
        def _():
            in_copy(g2, in0, si0).start()
        in_copy(g1, in1, si1).wait()

        @pl.when(k >= 2)
        def _():
            out_copy(grp(k - 1), fb1, so1).wait()
        transpose(in1, fb1)
        out_copy(g1, fb1, so1).start()

        @pl.when(k + 3 < NITER)
        def _():
            in_copy(g3, in1, si1).start()
        return carry

    lax.fori_loop(0, NITER // 2, body, 0)
    out_copy(grp(NITER - 2), fb0, so0).wait()
    out_copy(grp(NITER - 1), fb1, so1).wait()

    @pl.when(wid == NW - 1)
    def _():
        pltpu.sync_copy(tail_hbm, fb0.at[pl.ds(0, VTAIL * D)])
        pltpu.sync_copy(fb0.at[pl.ds(0, VTAIL * D)],
                        out_hbm.at[pl.ds(VFULL * D, VTAIL * D)])


@functools.partial(
    pl.kernel,
    mesh=_mesh,
    out_type=jax.ShapeDtypeStruct((B, D), jnp.float32),
    scratch_types=[
        pltpu.VMEM((BPW * L,), jnp.int32),
        pltpu.VMEM((GPI, D), jnp.float32),
        pltpu.VMEM((GPI, D), jnp.float32),
        pltpu.VMEM((GB, D), jnp.float32),
        pltpu.VMEM((GB, D), jnp.float32),
        pltpu.SemaphoreType.DMA,
        pltpu.SemaphoreType.DMA,
        pltpu.SemaphoreType.DMA,
        pltpu.SemaphoreType.DMA,
    ],
    compiler_params=pltpu.CompilerParams(use_tc_tiling_on_sc=False),
)
def _gather_sum(qf_hbm, table_hbm, tok_hbm, out_hbm, idx_v, rows0, rows1,
                stage0, stage1, sem0, sem1, semo0, semo1):
    del tok_hbm
    wid = lax.axis_index("s") * NC + lax.axis_index("c")
    base = wid * BPW
    NCOL = D // LANES

    pltpu.sync_copy(qf_hbm.at[pl.ds(base * L, BPW * L)], idx_v)

    def gather(g, buf, sem):
        return pltpu.make_async_copy(
            table_hbm.at[idx_v.at[pl.ds(g * GPI, GPI)]], buf, sem)

    def out_copy(g, stg, sem):
        return pltpu.make_async_copy(
            stg, out_hbm.at[pl.ds(base + g * GB, GB)], sem)

    def compute(g, buf, stg, semo):
        @pl.when(g >= 2)
        def _():
            out_copy(g - 2, stg, semo).wait()

        def body(l, accs):
            new = []
            for j in range(GB):
                r = j * L + l
                for c in range(NCOL):
                    new.append(accs[j * NCOL + c]
                               + buf[r, pl.ds(c * LANES, LANES)])
            return tuple(new)

        accs = lax.fori_loop(
            0, L, body,
            tuple(jnp.zeros((LANES,), jnp.float32)
                  for _ in range(GB * NCOL)))
        for j in range(GB):
            for c in range(NCOL):
                stg[j, pl.ds(c * LANES, LANES)] = accs[j * NCOL + c]
        out_copy(g, stg, semo).start()

    gather(0, rows0, sem0).start()

    def body(i, carry):
        g = 2 * i
        gather(g + 1, rows1, sem1).start()
        gather(g, rows0, sem0).wait()
        compute(g, rows0, stage0, semo0)

        @pl.when(g + 2 < NG)
        def _():
            gather(g + 2, rows0, sem0).start()

        gather(g + 1, rows1, sem1).wait()
        compute(g + 1, rows1, stage1, semo1)
        return carry

    lax.fori_loop(0, NG // 2, body, 0)

    out_copy(NG - 2, stage0, semo0).wait()
    out_copy(NG - 1, stage1, semo1).wait()


def _fence_body(x_ref, o_ref):
    o_ref[...] = x_ref[...]


def _fence(tl):
    return pl.pallas_call(
        _fence_body,
        grid=(1,),
        in_specs=[pl.BlockSpec((1024,), lambda i: (0,))],
        out_specs=pl.BlockSpec((1024,), lambda i: (0,)),
        out_shape=jax.ShapeDtypeStruct((1024,), jnp.float32),
    )(tl)


def _mm_body(x_ref, w_ref, o_ref):
    o_ref[...] = lax.dot_general(
        x_ref[...], w_ref[...],
        dimension_numbers=(((1,), (1,)), ((), ())),
        preferred_element_type=jnp.float32)


def _linear(x, w):
    return pl.pallas_call(
        _mm_body,
        grid=(8,),
        in_specs=[
            pl.BlockSpec((B // 8, D), lambda i: (i, 0)),
            pl.BlockSpec((D, D), lambda i: (0, 0)),
        ],
        out_specs=pl.BlockSpec((B // 8, D), lambda i: (i, 0)),
        out_shape=jax.ShapeDtypeStruct((B, D), jnp.float32),
    )(x, w)


def kernel(query, table, W):
    qf = jnp.reshape(query.astype(jnp.int32), (B * L,))
    tt = table.T
    tail = table[VFULL:, :].reshape(VTAIL * D)
    tl = _relayout(tt, tail)
    tok = _fence(tl)
    summed = _gather_sum(qf, tl.reshape(V, D), tok)
    return _linear(summed, W)


# speedup vs baseline: 1.3547x; 1.3547x over previous
import functools

import jax
import jax.numpy as jnp
from jax import lax
from jax.experimental import pallas as pl
from jax.experimental.pallas import tpu as pltpu
from jax.experimental.pallas import tpu_sc as plsc

V = 1_000_000
B = 16384
L = 50
D = 64
LANES = 16
NC = 2
NS = 16
NW = NC * NS

COLS = 128
FLAT = COLS * D
VFULL = (V // COLS) * COLS
NGRP = VFULL // COLS
NITER = 246
VTAIL = V - VFULL

BPW = B // NW
GB = 8
GPI = GB * L
NG = BPW // GB

_mesh = plsc.VectorSubcoreMesh(core_axis_name="c", subcore_axis_name="s")


@functools.partial(
    pl.kernel,
    mesh=_mesh,
    out_type=jax.ShapeDtypeStruct((V * D,), jnp.float32),
    scratch_types=[
        pltpu.VMEM((D, COLS), jnp.float32),
        pltpu.VMEM((D, COLS), jnp.float32),
        pltpu.VMEM((FLAT,), jnp.float32),
        pltpu.VMEM((FLAT,), jnp.float32),
        pltpu.SemaphoreType.DMA,
        pltpu.SemaphoreType.DMA,
        pltpu.SemaphoreType.DMA,
        pltpu.SemaphoreType.DMA,
    ],
    compiler_params=pltpu.CompilerParams(
        use_tc_tiling_on_sc=True, needs_layout_passes=False),
)
def _relayout(tt_hbm, tail_hbm, out_hbm, in0, in1, fb0, fb1, si0, si1, so0, so1):
    wid = lax.axis_index("s") * NC + lax.axis_index("c")

    def grp(k):
        return jnp.minimum(wid + k * NW, NGRP - 1)

    def in_copy(g, buf, sem):
        return pltpu.make_async_copy(
            tt_hbm.at[:, pl.ds(g * COLS, COLS)], buf, sem)

    def out_copy(g, fb, sem):
        return pltpu.make_async_copy(
            fb, out_hbm.at[pl.ds(g * FLAT, FLAT)], sem)

    pats = [(lax.iota(jnp.int32, LANES) + ch * LANES) * D
            for ch in range(COLS // LANES)]

    def transpose(buf, fb):
        @plsc.parallel_loop(0, D, unroll=8)
        def rbody(r):
            for ch in range(COLS // LANES):
                val = buf[r, pl.ds(ch * LANES, LANES)]
                plsc.store_scatter(fb, [pats[ch] + r], val)

    in_copy(grp(0), in0, si0).start()
    in_copy(grp(1), in1, si1).start()

    def body(i, carry):
        k = 2 * i
        g0, g1, g2, g3 = grp(k), grp(k + 1), grp(k + 2), grp(k + 3)
        in_copy(g0, in0, si0).wait()

        @pl.when(k >= 2)
        def _():
            out_copy(grp(k - 2), fb0, so0).wait()
        transpose(in0, fb0)
        out_copy(g0, fb0, so0).start()

        @pl.when(k + 2 < NITER)
        def _():
            in_copy(g2, in0, si0).start()
        in_copy(g1, in1, si1).wait()

        @pl.when(k >= 2)
        def _():
            out_copy(grp(k - 1), fb1, so1).wait()
        transpose(in1, fb1)
        out_copy(g1, fb1, so1).start()

        @pl.when(k + 3 < NITER)
        def _():
            in_copy(g3, in1, si1).start()
        return carry

    lax.fori_loop(0, NITER // 2, body, 0)
    out_copy(grp(NITER - 2), fb0, so0).wait()
    out_copy(grp(NITER - 1), fb1, so1).wait()

    @pl.when(wid == NW - 1)
    def _():
        pltpu.sync_copy(tail_hbm, fb0.at[pl.ds(0, VTAIL * D)])
        pltpu.sync_copy(fb0.at[pl.ds(0, VTAIL * D)],
                        out_hbm.at[pl.ds(VFULL * D, VTAIL * D)])


@functools.partial(
    pl.kernel,
    mesh=_mesh,
    out_type=jax.ShapeDtypeStruct((B, D), jnp.float32),
    scratch_types=[
        pltpu.VMEM((BPW * L,), jnp.int32),
        pltpu.VMEM((GPI, D), jnp.float32),
        pltpu.VMEM((GPI, D), jnp.float32),
        pltpu.VMEM((GB, D), jnp.float32),
        pltpu.VMEM((GB, D), jnp.float32),
        pltpu.SemaphoreType.DMA,
        pltpu.SemaphoreType.DMA,
        pltpu.SemaphoreType.DMA,
        pltpu.SemaphoreType.DMA,
    ],
    compiler_params=pltpu.CompilerParams(use_tc_tiling_on_sc=False),
)
def _gather_sum(qf_hbm, table_hbm, tok_hbm, out_hbm, idx_v, rows0, rows1,
                stage0, stage1, sem0, sem1, semo0, semo1):
    del tok_hbm
    wid = lax.axis_index("s") * NC + lax.axis_index("c")
    base = wid * BPW
    NCOL = D // LANES

    pltpu.sync_copy(qf_hbm.at[pl.ds(base * L, BPW * L)], idx_v)

    def gather(g, buf, sem):
        return pltpu.make_async_copy(
            table_hbm.at[idx_v.at[pl.ds(g * GPI, GPI)]], buf, sem)

    def out_copy(g, stg, sem):
        return pltpu.make_async_copy(
            stg, out_hbm.at[pl.ds(base + g * GB, GB)], sem)

    def compute(g, buf, stg, semo):
        @pl.when(g >= 2)
        def _():
            out_copy(g - 2, stg, semo).wait()

        def body(l, accs):
            new = []
            for j in range(GB):
                r = j * L + l
                for c in range(NCOL):
                    new.append(accs[j * NCOL + c]
                               + buf[r, pl.ds(c * LANES, LANES)])
            return tuple(new)

        accs = lax.fori_loop(
            0, L, body,
            tuple(jnp.zeros((LANES,), jnp.float32)
                  for _ in range(GB * NCOL)))
        for j in range(GB):
            for c in range(NCOL):
                stg[j, pl.ds(c * LANES, LANES)] = accs[j * NCOL + c]
        out_copy(g, stg, semo).start()

    gather(0, rows0, sem0).start()

    def body(i, carry):
        g = 2 * i
        gather(g + 1, rows1, sem1).start()
        gather(g, rows0, sem0).wait()
        compute(g, rows0, stage0, semo0)

        @pl.when(g + 2 < NG)
        def _():
            gather(g + 2, rows0, sem0).start()

        gather(g + 1, rows1, sem1).wait()
        compute(g + 1, rows1, stage1, semo1)
        return carry

    lax.fori_loop(0, NG // 2, body, 0)

    out_copy(NG - 2, stage0, semo0).wait()
    out_copy(NG - 1, stage1, semo1).wait()


def _fence_body(x_ref, o_ref):
    o_ref[...] = x_ref[...]


def _fence(tl):
    return pl.pallas_call(
        _fence_body,
        grid=(1,),
        in_specs=[pl.BlockSpec((1024,), lambda i: (0,))],
        out_specs=pl.BlockSpec((1024,), lambda i: (0,)),
        out_shape=jax.ShapeDtypeStruct((1024,), jnp.float32),
    )(tl)


def _mm_body(x_ref, w_ref, o_ref):
    o_ref[...] = lax.dot_general(
        x_ref[...], w_ref[...],
        dimension_numbers=(((1,), (1,)), ((), ())),
        preferred_element_type=jnp.float32)


def _linear(x, w):
    return pl.pallas_call(
        _mm_body,
        grid=(8,),
        in_specs=[
            pl.BlockSpec((B // 8, D), lambda i: (i, 0)),
            pl.BlockSpec((D, D), lambda i: (0, 0)),
        ],
        out_specs=pl.BlockSpec((B // 8, D), lambda i: (i, 0)),
        out_shape=jax.ShapeDtypeStruct((B, D), jnp.float32),
    )(x, w)


def kernel(query, table, W):
    qf = jnp.reshape(query.astype(jnp.int32), (B * L,))
    tt = table.T
    tail = table[VFULL:, :].reshape(VTAIL * D)
    tl = _relayout(tt, tail)
    tok = _fence(tl)
    summed = _gather_sum(qf, tl.reshape(V, D), tok)
    return _linear(summed, W)


# speedup vs baseline: 3.6716x; 2.7104x over previous
import functools

import jax
import jax.numpy as jnp
from jax import lax
from jax.experimental import pallas as pl
from jax.experimental.pallas import tpu as pltpu
from jax.experimental.pallas import tpu_sc as plsc

V = 1_000_000
B = 16384
L = 50
D = 64
LANES = 16
NC = 2
NS = 16
NW = NC * NS

COLS = 128
FLAT = COLS * D
VFULL = (V // COLS) * COLS
NGRP = VFULL // COLS
NITER = 246
VTAIL = V - VFULL

BPW = B // NW
GB = 8
GPI = GB * L
NG = BPW // GB

_mesh = plsc.VectorSubcoreMesh(core_axis_name="c", subcore_axis_name="s")


@functools.partial(
    pl.kernel,
    mesh=_mesh,
    out_type=jax.ShapeDtypeStruct((V * D,), jnp.float32),
    scratch_types=[
        pltpu.VMEM((D, COLS), jnp.float32),
        pltpu.VMEM((D, COLS), jnp.float32),
        pltpu.VMEM((FLAT,), jnp.float32),
        pltpu.VMEM((FLAT,), jnp.float32),
        pltpu.SemaphoreType.DMA,
        pltpu.SemaphoreType.DMA,
        pltpu.SemaphoreType.DMA,
        pltpu.SemaphoreType.DMA,
    ],
    compiler_params=pltpu.CompilerParams(
        use_tc_tiling_on_sc=True, needs_layout_passes=False),
)
def _relayout(tt_hbm, tail_hbm, out_hbm, in0, in1, fb0, fb1, si0, si1, so0, so1):
    wid = lax.axis_index("s") * NC + lax.axis_index("c")

    def grp(k):
        return jnp.minimum(wid + k * NW, NGRP - 1)

    def in_copy(g, buf, sem):
        return pltpu.make_async_copy(
            tt_hbm.at[:, pl.ds(g * COLS, COLS)], buf, sem)

    def out_copy(g, fb, sem):
        return pltpu.make_async_copy(
            fb, out_hbm.at[pl.ds(g * FLAT, FLAT)], sem)

    iota = lax.iota(jnp.int32, LANES)
    rot = [(iota + d) & (LANES - 1) for d in range(LANES)]
    rot64 = [rot[d] * D + iota for d in range(LANES)]

    def transpose(buf, fb):
        @plsc.parallel_loop(0, (D // LANES) * (COLS // LANES), unroll=2)
        def blk_body(blk):
            r0 = (blk >> 3) << 4
            l0 = (blk & 7) << 4
            row = iota + r0
            sbase = l0 * D + r0
            for d in range(LANES):
                val = plsc.load_gather(buf, [row, rot[d] + l0])
                plsc.store_scatter(fb, [rot64[d] + sbase], val)

    in_copy(grp(0), in0, si0).start()
    in_copy(grp(1), in1, si1).start()

    def body(i, carry):
        k = 2 * i
        g0, g1, g2, g3 = grp(k), grp(k + 1), grp(k + 2), grp(k + 3)
        in_copy(g0, in0, si0).wait()

        @pl.when(k >= 2)
        def _():
            out_copy(grp(k - 2), fb0, so0).wait()
        transpose(in0, fb0)
        out_copy(g0, fb0, so0).start()

        @pl.when(k + 2 < NITER)
        def _():
            in_copy(g2, in0, si0).start()
        in_copy(g1, in1, si1).wait()

        @pl.when(k >= 2)
        def _():
            out_copy(grp(k - 1), fb1, so1).wait()
        transpose(in1, fb1)
        out_copy(g1, fb1, so1).start()

        @pl.when(k + 3 < NITER)
        def _():
            in_copy(g3, in1, si1).start()
        return carry

    lax.fori_loop(0, NITER // 2, body, 0)
    out_copy(grp(NITER - 2), fb0, so0).wait()
    out_copy(grp(NITER - 1), fb1, so1).wait()

    @pl.when(wid == NW - 1)
    def _():
        pltpu.sync_copy(tail_hbm, fb0.at[pl.ds(0, VTAIL * D)])
        pltpu.sync_copy(fb0.at[pl.ds(0, VTAIL * D)],
                        out_hbm.at[pl.ds(VFULL * D, VTAIL * D)])


@functools.partial(
    pl.kernel,
    mesh=_mesh,
    out_type=jax.ShapeDtypeStruct((B, D), jnp.float32),
    scratch_types=[
        pltpu.VMEM((BPW * L,), jnp.int32),
        pltpu.VMEM((GPI, D), jnp.float32),
        pltpu.VMEM((GPI, D), jnp.float32),
        pltpu.VMEM((GB, D), jnp.float32),
        pltpu.VMEM((GB, D), jnp.float32),
        pltpu.SemaphoreType.DMA,
        pltpu.SemaphoreType.DMA,
        pltpu.SemaphoreType.DMA,
        pltpu.SemaphoreType.DMA,
    ],
    compiler_params=pltpu.CompilerParams(use_tc_tiling_on_sc=False),
)
def _gather_sum(qf_hbm, table_hbm, tok_hbm, out_hbm, idx_v, rows0, rows1,
                stage0, stage1, sem0, sem1, semo0, semo1):
    del tok_hbm
    wid = lax.axis_index("s") * NC + lax.axis_index("c")
    base = wid * BPW
    NCOL = D // LANES

    pltpu.sync_copy(qf_hbm.at[pl.ds(base * L, BPW * L)], idx_v)

    def gather(g, buf, sem):
        return pltpu.make_async_copy(
            table_hbm.at[idx_v.at[pl.ds(g * GPI, GPI)]], buf, sem)

    def out_copy(g, stg, sem):
        return pltpu.make_async_copy(
            stg, out_hbm.at[pl.ds(base + g * GB, GB)], sem)

    def compute(g, buf, stg, semo):
        @pl.when(g >= 2)
        def _():
            out_copy(g - 2, stg, semo).wait()

        def body(l, accs):
            new = []
            for j in range(GB):
                r = j * L + l
                for c in range(NCOL):
                    new.append(accs[j * NCOL + c]
                               + buf[r, pl.ds(c * LANES, LANES)])
            return tuple(new)

        accs = lax.fori_loop(
            0, L, body,
            tuple(jnp.zeros((LANES,), jnp.float32)
                  for _ in range(GB * NCOL)))
        for j in range(GB):
            for c in range(NCOL):
                stg[j, pl.ds(c * LANES, LANES)] = accs[j * NCOL + c]
        out_copy(g, stg, semo).start()

    gather(0, rows0, sem0).start()

    def body(i, carry):
        g = 2 * i
        gather(g + 1, rows1, sem1).start()
        gather(g, rows0, sem0).wait()
        compute(g, rows0, stage0, semo0)

        @pl.when(g + 2 < NG)
        def _():
            gather(g + 2, rows0, sem0).start()

        gather(g + 1, rows1, sem1).wait()
        compute(g + 1, rows1, stage1, semo1)
        return carry

    lax.fori_loop(0, NG // 2, body, 0)

    out_copy(NG - 2, stage0, semo0).wait()
    out_copy(NG - 1, stage1, semo1).wait()


def _fence_body(x_ref, o_ref):
    o_ref[...] = x_ref[...]


def _fence(tl):
    return pl.pallas_call(
        _fence_body,
        grid=(1,),
        in_specs=[pl.BlockSpec((1024,), lambda i: (0,))],
        out_specs=pl.BlockSpec((1024,), lambda i: (0,)),
        out_shape=jax.ShapeDtypeStruct((1024,), jnp.float32),
    )(tl)


def _mm_body(x_ref, w_ref, o_ref):
    o_ref[...] = lax.dot_general(
        x_ref[...], w_ref[...],
        dimension_numbers=(((1,), (1,)), ((), ())),
        preferred_element_type=jnp.float32)


def _linear(x, w):
    return pl.pallas_call(
        _mm_body,
        grid=(8,),
        in_specs=[
            pl.BlockSpec((B // 8, D), lambda i: (i, 0)),
            pl.BlockSpec((D, D), lambda i: (0, 0)),
        ],
        out_specs=pl.BlockSpec((B // 8, D), lambda i: (i, 0)),
        out_shape=jax.ShapeDtypeStruct((B, D), jnp.float32),
    )(x, w)


def kernel(query, table, W):
    qf = jnp.reshape(query.astype(jnp.int32), (B * L,))
    tt = table.T
    tail = table[VFULL:, :].reshape(VTAIL * D)
    tl = _relayout(tt, tail)
    tok = _fence(tl)
    summed = _gather_sum(qf, tl.reshape(V, D), tok)
    return _linear(summed, W)
